# baseline (speedup 1.0000x reference)
import functools

import jax
import jax.numpy as jnp
from jax import lax
from jax.experimental import pallas as pl
from jax.experimental.pallas import tpu as pltpu
from jax.experimental.pallas import tpu_sc as plsc

N = 100000
E = 1600000
NW = 32
N_PAD = 100096
PERS = N_PAD // 16
ZROWS = 391
CHUNK = 128
GDEPTH = 8
SGRP = 16
NSG = 25
E_PAD = NW * NSG * SGRP * CHUNK

_mesh = plsc.VectorSubcoreMesh(
    core_axis_name="c", subcore_axis_name="s", num_cores=2, num_subcores=16)


def _make_sc_agg(C: int, do_gather: bool):
  @functools.partial(
      pl.kernel,
      out_type=jax.ShapeDtypeStruct((2, N_PAD, C), jnp.float32),
      mesh=_mesh,
      compiler_params=pltpu.CompilerParams(use_tc_tiling_on_sc=False),
      scratch_types=[
          pltpu.VMEM_SHARED((N_PAD, C), jnp.float32),
          pltpu.VMEM((SGRP, CHUNK), jnp.int32),
          pltpu.VMEM((SGRP, CHUNK), jnp.int32),
          pltpu.VMEM((GDEPTH, CHUNK, C), jnp.float32),
          pltpu.VMEM((ZROWS, C), jnp.float32),
          pltpu.SemaphoreType.DMA,
          pltpu.SemaphoreType.DMA,
      ],
  )
  def agg(*refs):
    if do_gather:
      (u_hbm, src_hbm, dst_hbm, zeros_hbm, out_hbm,
       acc, idx_s, idx_d, rows, stage, gsem, ssem) = refs
    else:
      (u_hbm, dst_hbm, zeros_hbm, out_hbm,
       acc, idx_s, idx_d, rows, stage, gsem, ssem) = refs
    c = lax.axis_index("c")
    s = lax.axis_index("s")
    wid = s * 2 + c

    pltpu.sync_copy(zeros_hbm, stage)
    base = s * PERS

    def zbody(i, carry):
      pltpu.sync_copy(stage, acc.at[pl.ds(base + i * ZROWS, ZROWS)])
      return carry
    lax.fori_loop(0, PERS // ZROWS, zbody, 0)
    if not do_gather:
      pltpu.sync_copy(u_hbm, rows.at[0])
    plsc.subcore_barrier()

    def sbody(sg, carry):
      if do_gather:
        pltpu.sync_copy(src_hbm.at[wid, sg], idx_s)
      pltpu.sync_copy(dst_hbm.at[wid, sg], idx_d)

      def gbody(q, carry2):
        q8 = q * GDEPTH
        if do_gather:
          gds = [
              pltpu.async_copy(u_hbm.at[idx_s.at[q8 + j]], rows.at[j], gsem)
              for j in range(GDEPTH)
          ]
          sds = []
          for j in range(GDEPTH):
            gds[j].wait()
            sds.append(pltpu.async_copy(
                rows.at[j], acc.at[idx_d.at[q8 + j]], ssem, add=True))
          for d in sds:
            d.wait()
        else:
          sds = [
              pltpu.async_copy(
                  rows.at[0], acc.at[idx_d.at[q8 + j]], ssem, add=True)
              for j in range(GDEPTH)
          ]
          for d in sds:
            d.wait()
        return carry2
      lax.fori_loop(0, SGRP // GDEPTH, gbody, 0)
      return carry
    lax.fori_loop(0, NSG, sbody, 0)
    plsc.subcore_barrier()

    def wbody(i, carry):
      pltpu.sync_copy(acc.at[pl.ds(base + i * ZROWS, ZROWS)], stage)
      pltpu.sync_copy(stage, out_hbm.at[c, pl.ds(base + i * ZROWS, ZROWS)])
      return carry
    lax.fori_loop(0, PERS // ZROWS, wbody, 0)

  return agg


_sc_deg = _make_sc_agg(16, False)
_sc_agg16 = _make_sc_agg(16, True)

FR = N_PAD // 8
BLK = 3128
_GRID = (FR // BLK,)


def _row_spec():
  return pl.BlockSpec((BLK, 128), lambda i: (i, 0))


def _pair_spec():
  return pl.BlockSpec((2, BLK, 128), lambda i: (0, i, 0))


def _full_spec(shape):
  return pl.BlockSpec(shape, lambda i: (0,) * len(shape))


def _tc_call(body, in_specs, out_cols):
  if isinstance(out_cols, tuple):
    out_shape = tuple(jax.ShapeDtypeStruct((FR, oc), jnp.float32)
                      for oc in out_cols)
    out_specs = tuple(pl.BlockSpec((BLK, oc), lambda i: (i, 0))
                      for oc in out_cols)
  else:
    out_shape = jax.ShapeDtypeStruct((FR, out_cols), jnp.float32)
    out_specs = pl.BlockSpec((BLK, out_cols), lambda i: (i, 0))
  return pl.pallas_call(body, grid=_GRID, in_specs=in_specs,
                        out_specs=out_specs, out_shape=out_shape)


def _tck0(dp, x16, o_ux, o_dis):
  deg = dp[0] + dp[1] + 1.0
  dis = lax.rsqrt(deg)
  o_dis[...] = dis
  o_ux[...] = x16[...] * dis


def _tck1(s1, ux, dis, W1bd, b1t, W2bd, o_u2):
  d = dis[...]
  t = (s1[0] + s1[1] + ux[...]) * d
  h1 = jnp.maximum(jnp.dot(t, W1bd[...],
                           preferred_element_type=jnp.float32) + b1t[...], 0.0)
  o_u2[...] = jnp.dot(h1, W2bd[...], preferred_element_type=jnp.float32) * d


def _tck2(s2, u2, dis, b2t, W3bd, o_u3):
  d = dis[...]
  h2 = jnp.maximum((s2[0] + s2[1] + u2[...]) * d + b2t[...], 0.0)
  o_u3[...] = jnp.dot(h2, W3bd[...], preferred_element_type=jnp.float32) * d


def _tck3(s3, u3, dis, b3t, Wfbd, bft, o_y):
  d = dis[...]
  h3 = jnp.maximum((s3[0] + s3[1] + u3[...]) * d + b3t[...], 0.0)
  o_y[...] = jnp.dot(h3, Wfbd[...],
                     preferred_element_type=jnp.float32) + bft[...]


def kernel(x, edge_index, W1, b1, W2, b2, W3, b3, Wf, bf):
  f32 = jnp.float32
  eye8 = jnp.eye(8, dtype=f32)
  x16 = jnp.pad(x, ((0, N_PAD - N), (0, 13))).reshape(FR, 128)
  src = jnp.pad(edge_index[0], (0, E_PAD - E), constant_values=N)
  dst = jnp.pad(edge_index[1], (0, E_PAD - E), constant_values=N)
  srcr = src.reshape(NW, NSG, SGRP, CHUNK)
  dstr = dst.reshape(NW, NSG, SGRP, CHUNK)
  zeros16 = jnp.zeros((ZROWS, 16), f32)
  ones = jnp.ones((CHUNK, 16), f32)
  W1p = jnp.pad(W1, ((0, 13), (0, 0)))
  W3p = jnp.pad(W3, ((0, 0), (0, 8)))
  Wfp = jnp.pad(Wf, ((0, 8), (0, 0)))
  W1bd = jnp.kron(eye8, W1p)
  W2bd = jnp.kron(eye8, W2)
  W3bd = jnp.kron(eye8, W3p)
  Wfbd = jnp.kron(eye8, Wfp)
  b1t = jnp.tile(b1, 8).reshape(1, 256)
  b2t = jnp.tile(b2, 8).reshape(1, 128)
  b3t = jnp.tile(jnp.pad(b3, (0, 8)), 8).reshape(1, 128)
  bft = jnp.tile(bf, 8).reshape(1, 8)

  def unfold(a):
    return a.reshape(N_PAD, 16)

  dpart = _sc_deg(ones, dstr, zeros16).reshape(2, FR, 128)
  ux, dis = _tc_call(
      _tck0, [_pair_spec(), _row_spec()], (128, 128))(dpart, x16)

  s1 = _sc_agg16(unfold(ux), srcr, dstr, zeros16).reshape(2, FR, 128)
  u2 = _tc_call(
      _tck1,
      [_pair_spec(), _row_spec(), _row_spec(),
       _full_spec((128, 256)), _full_spec((1, 256)), _full_spec((256, 128))],
      128)(s1, ux, dis, W1bd, b1t, W2bd)

  s2 = _sc_agg16(unfold(u2), srcr, dstr, zeros16).reshape(2, FR, 128)
  u3 = _tc_call(
      _tck2,
      [_pair_spec(), _row_spec(), _row_spec(),
       _full_spec((1, 128)), _full_spec((128, 128))],
      128)(s2, u2, dis, b2t, W3bd)

  s3 = _sc_agg16(unfold(u3), srcr, dstr, zeros16).reshape(2, FR, 128)
  y8 = _tc_call(
      _tck3,
      [_pair_spec(), _row_spec(), _row_spec(),
       _full_spec((1, 128)), _full_spec((128, 8)), _full_spec((1, 8))],
      8)(s3, u3, dis, b3t, Wfbd, bft)

  return y8.reshape(N_PAD, 1)[:N]


# speedup vs baseline: 1.5330x; 1.5330x over previous
import functools

import jax
import jax.numpy as jnp
from jax import lax
from jax.experimental import pallas as pl
from jax.experimental.pallas import tpu as pltpu
from jax.experimental.pallas import tpu_sc as plsc

N = 100000
E = 1600000
NW = 32
N_PAD = 100096
PERS = N_PAD // 16
ZROWS = 782
CHUNK = 128
GDEPTH = 4
SGRP = 28
NSG = 14
E_PAD = NW * NSG * SGRP * CHUNK

_mesh = plsc.VectorSubcoreMesh(
    core_axis_name="c", subcore_axis_name="s", num_cores=2, num_subcores=16)


def _make_sc_agg(C: int, do_gather: bool):
  @functools.partial(
      pl.kernel,
      out_type=jax.ShapeDtypeStruct((2, N_PAD, C), jnp.float32),
      mesh=_mesh,
      compiler_params=pltpu.CompilerParams(use_tc_tiling_on_sc=False),
      scratch_types=[
          pltpu.VMEM_SHARED((N_PAD, C), jnp.float32),
          pltpu.VMEM((SGRP, CHUNK), jnp.int32),
          pltpu.VMEM((SGRP, CHUNK), jnp.int32),
          pltpu.VMEM((GDEPTH, CHUNK, C), jnp.float32),
          pltpu.VMEM((ZROWS, C), jnp.float32),
          pltpu.SemaphoreType.DMA,
          pltpu.SemaphoreType.DMA,
      ],
  )
  def agg(*refs):
    if do_gather:
      (u_hbm, src_hbm, dst_hbm, zeros_hbm, out_hbm,
       acc, idx_s, idx_d, rows, stage, gsem, ssem) = refs
    else:
      (u_hbm, dst_hbm, zeros_hbm, out_hbm,
       acc, idx_s, idx_d, rows, stage, gsem, ssem) = refs
    c = lax.axis_index("c")
    s = lax.axis_index("s")
    wid = s * 2 + c

    pltpu.sync_copy(zeros_hbm, stage)
    base = s * PERS

    def zbody(i, carry):
      pltpu.sync_copy(stage, acc.at[pl.ds(base + i * ZROWS, ZROWS)])
      return carry
    lax.fori_loop(0, PERS // ZROWS, zbody, 0)
    if not do_gather:
      pltpu.sync_copy(u_hbm, rows.at[0])
    plsc.subcore_barrier()

    def sbody(sg, carry):
      if do_gather:
        pltpu.sync_copy(src_hbm.at[wid, sg], idx_s)
      pltpu.sync_copy(dst_hbm.at[wid, sg], idx_d)

      def gbody(q, carry2):
        q8 = q * GDEPTH
        if do_gather:
          gds = [
              pltpu.async_copy(u_hbm.at[idx_s.at[q8 + j]], rows.at[j], gsem)
              for j in range(GDEPTH)
          ]
          for j in range(GDEPTH):
            gds[j].wait()
            pltpu.sync_copy(rows.at[j], acc.at[idx_d.at[q8 + j]], add=True)
        else:
          for j in range(GDEPTH):
            pltpu.sync_copy(rows.at[0], acc.at[idx_d.at[q8 + j]], add=True)
        return carry2
      lax.fori_loop(0, SGRP // GDEPTH, gbody, 0)
      return carry
    lax.fori_loop(0, NSG, sbody, 0)
    plsc.subcore_barrier()

    def wbody(i, carry):
      pltpu.sync_copy(acc.at[pl.ds(base + i * ZROWS, ZROWS)], stage)
      pltpu.sync_copy(stage, out_hbm.at[c, pl.ds(base + i * ZROWS, ZROWS)])
      return carry
    lax.fori_loop(0, PERS // ZROWS, wbody, 0)

  return agg


_sc_deg = _make_sc_agg(16, False)
_sc_agg16 = _make_sc_agg(16, True)

FR = N_PAD // 8
BLK = 3128
_GRID = (FR // BLK,)


def _row_spec():
  return pl.BlockSpec((BLK, 128), lambda i: (i, 0))


def _pair_spec():
  return pl.BlockSpec((2, BLK, 128), lambda i: (0, i, 0))


def _full_spec(shape):
  return pl.BlockSpec(shape, lambda i: (0,) * len(shape))


def _tc_call(body, in_specs, out_cols):
  if isinstance(out_cols, tuple):
    out_shape = tuple(jax.ShapeDtypeStruct((FR, oc), jnp.float32)
                      for oc in out_cols)
    out_specs = tuple(pl.BlockSpec((BLK, oc), lambda i: (i, 0))
                      for oc in out_cols)
  else:
    out_shape = jax.ShapeDtypeStruct((FR, out_cols), jnp.float32)
    out_specs = pl.BlockSpec((BLK, out_cols), lambda i: (i, 0))
  return pl.pallas_call(body, grid=_GRID, in_specs=in_specs,
                        out_specs=out_specs, out_shape=out_shape)


def _tck0(dp, x16, o_ux, o_dis):
  deg = dp[0] + dp[1] + 1.0
  dis = lax.rsqrt(deg)
  o_dis[...] = dis
  o_ux[...] = x16[...] * dis


def _tck1(s1, ux, dis, W1bd, b1t, W2bd, o_u2):
  d = dis[...]
  t = (s1[0] + s1[1] + ux[...]) * d
  h1 = jnp.maximum(jnp.dot(t, W1bd[...],
                           preferred_element_type=jnp.float32) + b1t[...], 0.0)
  o_u2[...] = jnp.dot(h1, W2bd[...], preferred_element_type=jnp.float32) * d


def _tck2(s2, u2, dis, b2t, W3bd, o_u3):
  d = dis[...]
  h2 = jnp.maximum((s2[0] + s2[1] + u2[...]) * d + b2t[...], 0.0)
  o_u3[...] = jnp.dot(h2, W3bd[...], preferred_element_type=jnp.float32) * d


def _tck3(s3, u3, dis, b3t, Wfbd, bft, o_y):
  d = dis[...]
  h3 = jnp.maximum((s3[0] + s3[1] + u3[...]) * d + b3t[...], 0.0)
  o_y[...] = jnp.dot(h3, Wfbd[...],
                     preferred_element_type=jnp.float32) + bft[...]


def kernel(x, edge_index, W1, b1, W2, b2, W3, b3, Wf, bf):
  f32 = jnp.float32
  eye8 = jnp.eye(8, dtype=f32)
  x16 = jnp.pad(x, ((0, N_PAD - N), (0, 13))).reshape(FR, 128)
  src = jnp.pad(edge_index[0], (0, E_PAD - E), constant_values=N)
  dst = jnp.pad(edge_index[1], (0, E_PAD - E), constant_values=N)
  srcr = src.reshape(NW, NSG, SGRP, CHUNK)
  dstr = dst.reshape(NW, NSG, SGRP, CHUNK)
  zeros16 = jnp.zeros((ZROWS, 16), f32)
  ones = jnp.ones((CHUNK, 16), f32)
  W1p = jnp.pad(W1, ((0, 13), (0, 0)))
  W3p = jnp.pad(W3, ((0, 0), (0, 8)))
  Wfp = jnp.pad(Wf, ((0, 8), (0, 0)))
  W1bd = jnp.kron(eye8, W1p)
  W2bd = jnp.kron(eye8, W2)
  W3bd = jnp.kron(eye8, W3p)
  Wfbd = jnp.kron(eye8, Wfp)
  b1t = jnp.tile(b1, 8).reshape(1, 256)
  b2t = jnp.tile(b2, 8).reshape(1, 128)
  b3t = jnp.tile(jnp.pad(b3, (0, 8)), 8).reshape(1, 128)
  bft = jnp.tile(bf, 8).reshape(1, 8)

  def unfold(a):
    return a.reshape(N_PAD, 16)

  dpart = _sc_deg(ones, dstr, zeros16).reshape(2, FR, 128)
  ux, dis = _tc_call(
      _tck0, [_pair_spec(), _row_spec()], (128, 128))(dpart, x16)

  s1 = _sc_agg16(unfold(ux), srcr, dstr, zeros16).reshape(2, FR, 128)
  u2 = _tc_call(
      _tck1,
      [_pair_spec(), _row_spec(), _row_spec(),
       _full_spec((128, 256)), _full_spec((1, 256)), _full_spec((256, 128))],
      128)(s1, ux, dis, W1bd, b1t, W2bd)

  s2 = _sc_agg16(unfold(u2), srcr, dstr, zeros16).reshape(2, FR, 128)
  u3 = _tc_call(
      _tck2,
      [_pair_spec(), _row_spec(), _row_spec(),
       _full_spec((1, 128)), _full_spec((128, 128))],
      128)(s2, u2, dis, b2t, W3bd)

  s3 = _sc_agg16(unfold(u3), srcr, dstr, zeros16).reshape(2, FR, 128)
  y8 = _tc_call(
      _tck3,
      [_pair_spec(), _row_spec(), _row_spec(),
       _full_spec((1, 128)), _full_spec((128, 8)), _full_spec((1, 8))],
      8)(s3, u3, dis, b3t, Wfbd, bft)

  return y8.reshape(N_PAD, 1)[:N]


# speedup vs baseline: 1.5471x; 1.0092x over previous
import functools

import jax
import jax.numpy as jnp
from jax import lax
from jax.experimental import pallas as pl
from jax.experimental.pallas import tpu as pltpu
from jax.experimental.pallas import tpu_sc as plsc

N = 100000
E = 1600000
NW = 32
N_PAD = 100096
PERS = N_PAD // 16
ZROWS = 782
CHUNK = 128
GDEPTH = 4
SGRP = 28
NSG = 14
E_PAD = NW * NSG * SGRP * CHUNK

_mesh = plsc.VectorSubcoreMesh(
    core_axis_name="c", subcore_axis_name="s", num_cores=2, num_subcores=16)


def _make_sc_agg(C: int, do_gather: bool):
  @functools.partial(
      pl.kernel,
      out_type=jax.ShapeDtypeStruct((2, N_PAD, C), jnp.float32),
      mesh=_mesh,
      compiler_params=pltpu.CompilerParams(use_tc_tiling_on_sc=False),
      scratch_types=[
          pltpu.VMEM_SHARED((N_PAD, C), jnp.float32),
          pltpu.VMEM((SGRP, CHUNK), jnp.int32),
          pltpu.VMEM((SGRP, CHUNK), jnp.int32),
          pltpu.VMEM((GDEPTH, CHUNK, C), jnp.float32),
          pltpu.VMEM((ZROWS, C), jnp.float32),
          pltpu.SemaphoreType.DMA,
          pltpu.SemaphoreType.DMA,
      ],
  )
  def agg(*refs):
    if do_gather:
      (u_hbm, src_hbm, dst_hbm, zeros_hbm, out_hbm,
       acc, idx_s, idx_d, rows, stage, gsem, ssem) = refs
    else:
      (u_hbm, dst_hbm, zeros_hbm, out_hbm,
       acc, idx_s, idx_d, rows, stage, gsem, ssem) = refs
    c = lax.axis_index("c")
    s = lax.axis_index("s")
    wid = s * 2 + c

    pltpu.sync_copy(zeros_hbm, stage)
    base = s * PERS

    def zbody(i, carry):
      pltpu.sync_copy(stage, acc.at[pl.ds(base + i * ZROWS, ZROWS)])
      return carry
    lax.fori_loop(0, PERS // ZROWS, zbody, 0)
    if not do_gather:
      pltpu.sync_copy(u_hbm, rows.at[0])
    plsc.subcore_barrier()

    def sbody(sg, carry):
      if do_gather:
        pltpu.sync_copy(src_hbm.at[wid, sg], idx_s)
      pltpu.sync_copy(dst_hbm.at[wid, sg], idx_d)

      def gbody(q, carry2):
        q8 = q * GDEPTH
        if do_gather:
          gds = [
              pltpu.async_copy(u_hbm.at[idx_s.at[q8 + j]], rows.at[j], gsem)
              for j in range(GDEPTH)
          ]
          for j in range(GDEPTH):
            gds[j].wait()
            pltpu.sync_copy(rows.at[j], acc.at[idx_d.at[q8 + j]], add=True)
        else:
          for j in range(GDEPTH):
            pltpu.sync_copy(rows.at[0], acc.at[idx_d.at[q8 + j]], add=True)
        return carry2
      lax.fori_loop(0, SGRP // GDEPTH, gbody, 0)
      return carry
    lax.fori_loop(0, NSG, sbody, 0)
    plsc.subcore_barrier()

    def wbody(i, carry):
      pltpu.sync_copy(acc.at[pl.ds(base + i * ZROWS, ZROWS)], stage)
      pltpu.sync_copy(stage, out_hbm.at[c, pl.ds(base + i * ZROWS, ZROWS)])
      return carry
    lax.fori_loop(0, PERS // ZROWS, wbody, 0)

  return agg


_sc_deg = _make_sc_agg(16, False)
_sc_agg16 = _make_sc_agg(16, True)

FR = N_PAD // 8
BLK = 3128
_GRID = (FR // BLK,)


def _row_spec():
  return pl.BlockSpec((BLK, 128), lambda i: (i, 0))


def _pair_spec():
  return pl.BlockSpec((2, BLK, 128), lambda i: (0, i, 0))


def _full_spec(shape):
  return pl.BlockSpec(shape, lambda i: (0,) * len(shape))


def _tc_call(body, in_specs, out_cols):
  if isinstance(out_cols, tuple):
    out_shape = tuple(jax.ShapeDtypeStruct((FR, oc), jnp.float32)
                      for oc in out_cols)
    out_specs = tuple(pl.BlockSpec((BLK, oc), lambda i: (i, 0))
                      for oc in out_cols)
  else:
    out_shape = jax.ShapeDtypeStruct((FR, out_cols), jnp.float32)
    out_specs = pl.BlockSpec((BLK, out_cols), lambda i: (i, 0))
  return pl.pallas_call(body, grid=_GRID, in_specs=in_specs,
                        out_specs=out_specs, out_shape=out_shape)


def _tck0(dp, x16, o_ux, o_dis):
  deg = dp[0] + dp[1] + 1.0
  dis = lax.rsqrt(deg)
  o_dis[...] = dis
  o_ux[...] = x16[...] * dis


def _tck1(s1, ux, dis, W1bd, b1t, W2bd, o_u2):
  d = dis[...]
  t = (s1[0] + s1[1] + ux[...]) * d
  h1 = jnp.maximum(jnp.dot(t, W1bd[...],
                           preferred_element_type=jnp.float32) + b1t[...], 0.0)
  o_u2[...] = jnp.dot(h1, W2bd[...], preferred_element_type=jnp.float32) * d


def _tck2(s2, u2, dis, b2t, W3bd, o_u3):
  d = dis[...]
  h2 = jnp.maximum((s2[0] + s2[1] + u2[...]) * d + b2t[...], 0.0)
  o_u3[...] = jnp.dot(h2, W3bd[...], preferred_element_type=jnp.float32) * d


def _tck3(s3, u3, dis, b3t, Wfbd, bft, o_y):
  d = dis[...]
  h3 = jnp.maximum((s3[0] + s3[1] + u3[...]) * d + b3t[...], 0.0)
  o_y[...] = jnp.dot(h3, Wfbd[...],
                     preferred_element_type=jnp.float32) + bft[...]


def kernel(x, edge_index, W1, b1, W2, b2, W3, b3, Wf, bf):
  f32 = jnp.float32
  eye8 = jnp.eye(8, dtype=f32)
  x16 = jnp.pad(x, ((0, N_PAD - N), (0, 13))).reshape(FR, 128)
  ei = jnp.pad(edge_index, ((0, 0), (0, E_PAD - E)),
               constant_values=N).reshape(2, NW, NSG, SGRP, CHUNK)
  srcr = ei[0]
  dstr = ei[1]
  zeros16 = jnp.zeros((ZROWS, 16), f32)
  ones = jnp.ones((CHUNK, 16), f32)
  W1p = jnp.pad(W1, ((0, 13), (0, 0)))
  W3p = jnp.pad(W3, ((0, 0), (0, 8)))
  Wfp = jnp.pad(Wf, ((0, 8), (0, 0)))
  W1bd = jnp.kron(eye8, W1p)
  W2bd = jnp.kron(eye8, W2)
  W3bd = jnp.kron(eye8, W3p)
  Wfbd = jnp.kron(eye8, Wfp)
  b1t = jnp.tile(b1, 8).reshape(1, 256)
  b2t = jnp.tile(b2, 8).reshape(1, 128)
  b3t = jnp.tile(jnp.pad(b3, (0, 8)), 8).reshape(1, 128)
  bft = jnp.tile(bf, 8).reshape(1, 8)

  def unfold(a):
    return a.reshape(N_PAD, 16)

  dpart = _sc_deg(ones, dstr, zeros16).reshape(2, FR, 128)
  ux, dis = _tc_call(
      _tck0, [_pair_spec(), _row_spec()], (128, 128))(dpart, x16)

  s1 = _sc_agg16(unfold(ux), srcr, dstr, zeros16).reshape(2, FR, 128)
  u2 = _tc_call(
      _tck1,
      [_pair_spec(), _row_spec(), _row_spec(),
       _full_spec((128, 256)), _full_spec((1, 256)), _full_spec((256, 128))],
      128)(s1, ux, dis, W1bd, b1t, W2bd)

  s2 = _sc_agg16(unfold(u2), srcr, dstr, zeros16).reshape(2, FR, 128)
  u3 = _tc_call(
      _tck2,
      [_pair_spec(), _row_spec(), _row_spec(),
       _full_spec((1, 128)), _full_spec((128, 128))],
      128)(s2, u2, dis, b2t, W3bd)

  s3 = _sc_agg16(unfold(u3), srcr, dstr, zeros16).reshape(2, FR, 128)
  y8 = _tc_call(
      _tck3,
      [_pair_spec(), _row_spec(), _row_spec(),
       _full_spec((1, 128)), _full_spec((128, 8)), _full_spec((1, 8))],
      8)(s3, u3, dis, b3t, Wfbd, bft)

  return y8.reshape(N_PAD, 1)[:N]
